# baseline (speedup 1.0000x reference)
import jax
import jax.numpy as jnp
from jax import lax
from jax.experimental import pallas as pl
from jax.experimental.pallas import tpu as pltpu
from jax.experimental.pallas import tpu_sc as plsc

F32 = jnp.float32


def _rup(v, m):
    return (v + m - 1) // m * m


def kernel(x, edge_index, conv1_W, conv1_b, conv2_W, conv2_b, conv3_W, conv3_b,
           conv4_W, conv4_b, bn1_g, bn1_b, bn2_g, bn2_b, bn3_g, bn3_b,
           lin_W, lin_b):
    N = x.shape[0]
    E = edge_index.shape[1]
    H = conv1_W.shape[2]
    HW = H // 2
    RPT = _rup(-(-N // 16), 8)
    NP = 16 * RPT
    C = 400
    EPT = E // 16
    assert EPT % C == 0 and C % 8 == 0

    mesh = plsc.VectorSubcoreMesh(core_axis_name="c", subcore_axis_name="s")


    NCH = EPT // C
    assert NCH % 2 == 1
    NPAIR = (NCH - 1) // 2

    def _sc_prop():
        def body(eidx_hbm, flg_hbm, u0_hbm, u1_hbm, zer_hbm, o0_hbm, o1_hbm,
                 eb0, eb1, rows0, rows1, acc, fv, sem0, sem1):
            c = lax.axis_index("c")
            s = lax.axis_index("s")
            r0 = s * RPT
            base0 = s * EPT
            pltpu.sync_copy(flg_hbm, fv)
            rsel = fv[...][0]

            def run(u_hbm, o_hbm):
                pltpu.sync_copy(zer_hbm, acc.at[pl.ds(r0, RPT), :])
                plsc.subcore_barrier()

                def load_idx(k, eb):
                    pltpu.sync_copy(
                        eidx_hbm.at[:, pl.ds(base0 + k * C, C)], eb)

                    @pl.when(rsel == 0)
                    def _():
                        pltpu.sync_copy(
                            eidx_hbm.at[0, pl.ds(base0 + k * C, C)], eb.at[1])

                def gather(eb, rows, sem):
                    pltpu.async_copy(u_hbm.at[eb.at[0]], rows, sem)

                def gwait(eb, rows, sem):
                    pltpu.make_async_copy(u_hbm.at[eb.at[0]], rows, sem).wait()

                def scatter(eb, rows):
                    pltpu.sync_copy(rows, acc.at[eb.at[1]], add=True)

                load_idx(0, eb0)
                gather(eb0, rows0, sem0)
                load_idx(1, eb1)

                def pair(i, carry):
                    a = 2 * i + 1
                    gather(eb1, rows1, sem1)
                    gwait(eb0, rows0, sem0)
                    scatter(eb0, rows0)
                    load_idx(a + 1, eb0)
                    gather(eb0, rows0, sem0)
                    gwait(eb1, rows1, sem1)
                    scatter(eb1, rows1)

                    @pl.when(a + 2 < NCH)
                    def _():
                        load_idx(a + 2, eb1)

                    return carry

                lax.fori_loop(0, NPAIR, pair, 0)
                gwait(eb0, rows0, sem0)
                scatter(eb0, rows0)
                plsc.subcore_barrier()
                pltpu.sync_copy(acc.at[pl.ds(r0, RPT), :],
                                o_hbm.at[pl.ds(r0, RPT), :])

            @pl.when(c == 0)
            def _():
                run(u0_hbm, o0_hbm)

            @pl.when(c == 1)
            def _():
                run(u1_hbm, o1_hbm)

        return pl.kernel(
            body,
            out_type=[jax.ShapeDtypeStruct((NP, HW), F32)] * 2,
            mesh=mesh,
            compiler_params=pltpu.CompilerParams(use_tc_tiling_on_sc=False),
            scratch_types=[
                pltpu.VMEM((2, C), jnp.int32),
                pltpu.VMEM((2, C), jnp.int32),
                pltpu.VMEM((C, HW), F32),
                pltpu.VMEM((C, HW), F32),
                pltpu.VMEM_SHARED((NP, HW), F32),
                pltpu.VMEM((16,), jnp.int32),
                pltpu.SemaphoreType.DMA,
                pltpu.SemaphoreType.DMA,
            ],
        )


    RB = 2 * RPT

    def _rows(ncols):
        return pl.BlockSpec((RB, ncols), lambda i: (i, 0))

    def _bcast(shape):
        return pl.BlockSpec(shape, lambda i: tuple(0 for _ in shape))

    def _call(body, in_specs, out_specs, out_shape):
        return pl.pallas_call(body, grid=(NP // RB,), in_specs=in_specs,
                              out_specs=out_specs, out_shape=out_shape)

    def _zn(z_ref, st_ref, g_ref, bb_ref):
        stv = st_ref[...]
        m = stv[0:1, :] / N
        v = stv[1:2, :] / N - m * m
        rstd = lax.rsqrt(v + 1e-5)
        return (z_ref[...] - m) * rstd * g_ref[...] + bb_ref[...]

    def _stats(i, z, st_ref):
        gid = i * RB + lax.broadcasted_iota(jnp.int32, (RB, 1), 0)
        zm = jnp.where(gid < N, z, 0.0)
        ssum = jnp.sum(zm, axis=0, keepdims=True)
        ssq = jnp.sum(zm * zm, axis=0, keepdims=True)
        upd = jnp.concatenate([ssum, ssq, jnp.zeros((6, H), F32)], axis=0)

        @pl.when(i == 0)
        def _():
            st_ref[...] = jnp.zeros((8, H), F32)

        st_ref[...] += upd

    def tc0(o0, xin):
        def body(o0_ref, x_ref, dis_o, u_o):
            deg = o0_ref[:, 0:1]
            dis = jnp.where(deg > 0, lax.rsqrt(jnp.maximum(deg, 1.0)), 0.0)
            dis_o[...] = dis
            ux = dis * x_ref[...]
            u_o[...] = jnp.concatenate(
                [ux, jnp.zeros((RB, HW - 3), F32)], axis=1)

        return _call(body, [_rows(HW), _rows(3)],
                     [_rows(1), _rows(HW)],
                     [jax.ShapeDtypeStruct((NP, 1), F32),
                      jax.ShapeDtypeStruct((NP, HW), F32)])(o0, xin)

    def tc_m1(o0, dis):
        def body(o0_ref, dis_ref, u_o):
            w = -(dis_ref[...] * dis_ref[...])
            u_o[...] = w * o0_ref[...]

        return _call(body, [_rows(HW), _rows(1)], [_rows(HW)],
                     [jax.ShapeDtypeStruct((NP, HW), F32)])(o0, dis)[0]

    def tc_b1(xin, a0, q0, dis, W, b):
        def body(x_ref, a0_ref, q0_ref, dis_ref, w_ref, b_ref, z_o, st_o):
            i = pl.program_id(0)
            xb = x_ref[...]
            dis_ = dis_ref[...]
            t1 = -dis_ * a0_ref[:, :3]
            t2 = -2.0 * dis_ * q0_ref[:, :3] - xb
            Wm = w_ref[...]
            z = (jnp.dot(xb, Wm[0], preferred_element_type=F32)
                 + jnp.dot(t1, Wm[1], preferred_element_type=F32)
                 + jnp.dot(t2, Wm[2], preferred_element_type=F32)
                 + b_ref[...])
            z = jnp.where(z > 0, z, 0.01 * z)
            z_o[...] = z
            _stats(i, z, st_o)

        return _call(body,
                     [_rows(3), _rows(HW), _rows(HW), _rows(1),
                      _bcast((3, 3, H)), _bcast((1, H))],
                     [_rows(H), _bcast((8, H))],
                     [jax.ShapeDtypeStruct((NP, H), F32),
                      jax.ShapeDtypeStruct((8, H), F32)])(
                         xin, a0, q0, dis, W, b)

    def tc_a(z, st, g, bb, dis):
        def body(z_ref, st_ref, g_ref, bb_ref, dis_ref, u0_o, u1_o):
            zn = _zn(z_ref, st_ref, g_ref, bb_ref)
            un = dis_ref[...] * zn
            u0_o[...] = un[:, :HW]
            u1_o[...] = un[:, HW:]

        return _call(body,
                     [_rows(H), _bcast((8, H)), _bcast((1, H)), _bcast((1, H)),
                      _rows(1)],
                     [_rows(HW), _rows(HW)],
                     [jax.ShapeDtypeStruct((NP, HW), F32)] * 2)(
                         z, st, g, bb, dis)

    def tc_m(c0, c1, dis):
        def body(c0_ref, c1_ref, dis_ref, o0_ref, o1_ref):
            w = -(dis_ref[...] * dis_ref[...])
            o0_ref[...] = w * c0_ref[...]
            o1_ref[...] = w * c1_ref[...]

        return _call(body, [_rows(HW), _rows(HW), _rows(1)],
                     [_rows(HW), _rows(HW)],
                     [jax.ShapeDtypeStruct((NP, HW), F32)] * 2)(c0, c1, dis)

    def tc_b(z, st, g, bb, c0, c1, d0, d1, dis, W, b, leaky):
        def body(z_ref, st_ref, g_ref, bb_ref, c0_ref, c1_ref, d0_ref, d1_ref,
                 dis_ref, w_ref, b_ref, z_o, st_o):
            i = pl.program_id(0)
            zn = _zn(z_ref, st_ref, g_ref, bb_ref)
            dis_ = dis_ref[...]
            a1 = jnp.concatenate([c0_ref[...], c1_ref[...]], axis=1)
            a2 = jnp.concatenate([d0_ref[...], d1_ref[...]], axis=1)
            t1 = -dis_ * a1
            t2 = -2.0 * dis_ * a2 - zn
            Wm = w_ref[...]
            zz = (jnp.dot(zn, Wm[0], preferred_element_type=F32)
                  + jnp.dot(t1, Wm[1], preferred_element_type=F32)
                  + jnp.dot(t2, Wm[2], preferred_element_type=F32)
                  + b_ref[...])
            if leaky:
                zz = jnp.where(zz > 0, zz, 0.01 * zz)
            else:
                zz = jnp.maximum(zz, 0.0)
            z_o[...] = zz
            _stats(i, zz, st_o)

        return _call(body,
                     [_rows(H), _bcast((8, H)), _bcast((1, H)), _bcast((1, H)),
                      _rows(HW), _rows(HW), _rows(HW), _rows(HW),
                      _rows(1), _bcast((3, H, H)), _bcast((1, H))],
                     [_rows(H), _bcast((8, H))],
                     [jax.ShapeDtypeStruct((NP, H), F32),
                      jax.ShapeDtypeStruct((8, H), F32)])(
                         z, st, g, bb, c0, c1, d0, d1, dis, W, b)

    def tc_b4(z, st, g, bb, c0, c1, d0, d1, dis, W, b, lw, lb):
        def body(z_ref, st_ref, g_ref, bb_ref, c0_ref, c1_ref, d0_ref, d1_ref,
                 dis_ref, w_ref, b_ref, lw_ref, lb_ref, out_o):
            zn = _zn(z_ref, st_ref, g_ref, bb_ref)
            dis_ = dis_ref[...]
            a1 = jnp.concatenate([c0_ref[...], c1_ref[...]], axis=1)
            a2 = jnp.concatenate([d0_ref[...], d1_ref[...]], axis=1)
            t1 = -dis_ * a1
            t2 = -2.0 * dis_ * a2 - zn
            Wm = w_ref[...]
            h = (jnp.dot(zn, Wm[0], preferred_element_type=F32)
                 + jnp.dot(t1, Wm[1], preferred_element_type=F32)
                 + jnp.dot(t2, Wm[2], preferred_element_type=F32)
                 + b_ref[...])
            nrm = jnp.sqrt(jnp.sum(h * h, axis=1, keepdims=True))
            xr = h / jnp.maximum(nrm, 1e-12)
            out_o[...] = lax.dot_general(
                xr, lw_ref[...], (((1,), (1,)), ((), ())),
                preferred_element_type=F32) + lb_ref[...]

        return _call(body,
                     [_rows(H), _bcast((8, H)), _bcast((1, H)), _bcast((1, H)),
                      _rows(HW), _rows(HW), _rows(HW), _rows(HW),
                      _rows(1), _bcast((3, H, H)), _bcast((1, H)),
                      _bcast((3, H)), _bcast((1, 3))],
                     [_rows(3)],
                     [jax.ShapeDtypeStruct((N, 3), F32)])(
                         z, st, g, bb, c0, c1, d0, d1, dis, W, b, lw, lb)[0]


    x = x.astype(F32)
    src = edge_index[0]
    dst = edge_index[1]
    ones_u = jnp.zeros((NP, HW), F32).at[:, 0].set(1.0)
    zer32 = jnp.zeros((RPT, HW), F32)
    zeros_u = jnp.zeros((NP, HW), F32)
    b1 = conv1_b.reshape(1, H)
    b2 = conv2_b.reshape(1, H)
    b3 = conv3_b.reshape(1, H)
    b4 = conv4_b.reshape(1, H)
    g1, gb1 = bn1_g.reshape(1, H), bn1_b.reshape(1, H)
    g2, gb2 = bn2_g.reshape(1, H), bn2_b.reshape(1, H)
    g3, gb3 = bn3_g.reshape(1, H), bn3_b.reshape(1, H)
    lbr = lin_b.reshape(1, 3)

    sc_prop = _sc_prop()

    e_norm = jnp.stack([src, dst])

    def loop_body(k, carry):
        u0, u1, a0, a1v, z, st, dis, out = carry
        flg = jnp.where(k == 0, jnp.zeros((16,), jnp.int32),
                        jnp.zeros((16,), jnp.int32).at[0].set(1))
        o0, o1 = sc_prop(e_norm, flg, u0, u1, zer32)

        def s0(u0, u1, a0, a1v, z, st, dis, out):
            dis2, un0 = tc0(o0, x)
            return un0, u1, a0, a1v, z, st, dis2, out

        def s1(u0, u1, a0, a1v, z, st, dis, out):
            un0 = tc_m1(o0, dis)
            return un0, u1, o0, a1v, z, st, dis, out

        def s2(u0, u1, a0, a1v, z, st, dis, out):
            z1, st1 = tc_b1(x, a0, o0, dis, conv1_W, b1)
            n0, n1 = tc_a(z1, st1, g1, gb1, dis)
            return n0, n1, a0, a1v, z1, st1, dis, out

        def even(u0, u1, a0, a1v, z, st, dis, out):
            n0, n1 = tc_m(o0, o1, dis)
            return n0, n1, o0, o1, z, st, dis, out

        def mk_odd(g, gb, W, b, gn, gbn, leaky):
            def odd(u0, u1, a0, a1v, z, st, dis, out):
                z2, st2 = tc_b(z, st, g, gb, a0, a1v, o0, o1, dis, W, b, leaky)
                n0, n1 = tc_a(z2, st2, gn, gbn, dis)
                return n0, n1, a0, a1v, z2, st2, dis, out
            return odd

        def last(u0, u1, a0, a1v, z, st, dis, out):
            out2 = tc_b4(z, st, g3, gb3, a0, a1v, o0, o1, dis,
                         conv4_W, b4, lin_W, lbr)
            return u0, u1, a0, a1v, z, st, dis, out2

        branches = (s0, s1, s2,
                    even, mk_odd(g1, gb1, conv2_W, b2, g2, gb2, True),
                    even, mk_odd(g2, gb2, conv3_W, b3, g3, gb3, False),
                    even, last)
        return lax.switch(k, branches, u0, u1, a0, a1v, z, st, dis, out)

    zv = jnp.zeros((NP, HW), F32)
    carry = (ones_u, zeros_u, zv, zv,
             jnp.zeros((NP, H), F32), jnp.zeros((8, H), F32),
             jnp.zeros((NP, 1), F32), jnp.zeros((N, 3), F32))
    res = lax.fori_loop(0, 9, loop_body, carry)
    return res[7]


# speedup vs baseline: 1.0520x; 1.0520x over previous
import jax
import jax.numpy as jnp
from jax import lax
from jax.experimental import pallas as pl
from jax.experimental.pallas import tpu as pltpu
from jax.experimental.pallas import tpu_sc as plsc

F32 = jnp.float32


def _rup(v, m):
    return (v + m - 1) // m * m


def kernel(x, edge_index, conv1_W, conv1_b, conv2_W, conv2_b, conv3_W, conv3_b,
           conv4_W, conv4_b, bn1_g, bn1_b, bn2_g, bn2_b, bn3_g, bn3_b,
           lin_W, lin_b):
    N = x.shape[0]
    E = edge_index.shape[1]
    H = conv1_W.shape[2]
    HW = H // 2
    RPT = _rup(-(-N // 16), 8)
    NP = 16 * RPT
    C = 400
    EPT = E // 16
    assert EPT % C == 0 and C % 8 == 0

    mesh = plsc.VectorSubcoreMesh(core_axis_name="c", subcore_axis_name="s")


    NCH = EPT // C
    assert NCH % 2 == 1
    NPAIR = (NCH - 1) // 2

    def _sc_prop():
        def body(eidx_hbm, flg_hbm, u0_hbm, u1_hbm, zer_hbm, o0_hbm, o1_hbm,
                 eb0, eb1, rows0, rows1, acc, fv, sem0, sem1):
            c = lax.axis_index("c")
            s = lax.axis_index("s")
            r0 = s * RPT
            base0 = s * EPT
            pltpu.sync_copy(flg_hbm, fv)
            rsel = fv[...][0]

            def run(u_hbm, o_hbm):
                pltpu.sync_copy(zer_hbm, acc.at[pl.ds(r0, RPT), :])
                plsc.subcore_barrier()

                def load_idx(k, eb):
                    pltpu.sync_copy(
                        eidx_hbm.at[:, pl.ds(base0 + k * C, C)], eb)

                    @pl.when(rsel == 0)
                    def _():
                        pltpu.sync_copy(
                            eidx_hbm.at[0, pl.ds(base0 + k * C, C)], eb.at[1])

                def gather(eb, rows, sem):
                    pltpu.async_copy(u_hbm.at[eb.at[0]], rows, sem)

                def gwait(eb, rows, sem):
                    pltpu.make_async_copy(u_hbm.at[eb.at[0]], rows, sem).wait()

                def scatter(eb, rows):
                    pltpu.sync_copy(rows, acc.at[eb.at[1]], add=True)

                load_idx(0, eb0)
                gather(eb0, rows0, sem0)
                load_idx(1, eb1)

                def pair(i, carry):
                    a = 2 * i + 1
                    gather(eb1, rows1, sem1)
                    gwait(eb0, rows0, sem0)
                    scatter(eb0, rows0)
                    load_idx(a + 1, eb0)
                    gather(eb0, rows0, sem0)
                    gwait(eb1, rows1, sem1)
                    scatter(eb1, rows1)

                    @pl.when(a + 2 < NCH)
                    def _():
                        load_idx(a + 2, eb1)

                    return carry

                lax.fori_loop(0, NPAIR, pair, 0)
                gwait(eb0, rows0, sem0)
                scatter(eb0, rows0)
                plsc.subcore_barrier()
                pltpu.sync_copy(acc.at[pl.ds(r0, RPT), :],
                                o_hbm.at[pl.ds(r0, RPT), :])

            @pl.when(c == 0)
            def _():
                run(u0_hbm, o0_hbm)

            @pl.when(c == 1)
            def _():
                run(u1_hbm, o1_hbm)

        return pl.kernel(
            body,
            out_type=[jax.ShapeDtypeStruct((NP, HW), F32)] * 2,
            mesh=mesh,
            compiler_params=pltpu.CompilerParams(use_tc_tiling_on_sc=False),
            scratch_types=[
                pltpu.VMEM((2, C), jnp.int32),
                pltpu.VMEM((2, C), jnp.int32),
                pltpu.VMEM((C, HW), F32),
                pltpu.VMEM((C, HW), F32),
                pltpu.VMEM_SHARED((NP, HW), F32),
                pltpu.VMEM((16,), jnp.int32),
                pltpu.SemaphoreType.DMA,
                pltpu.SemaphoreType.DMA,
            ],
        )


    def _rows(ncols):
        return pl.BlockSpec((RPT, ncols), lambda i: (i, 0))

    def _bcast(shape):
        return pl.BlockSpec(shape, lambda i: tuple(0 for _ in shape))

    def _call(body, in_specs, out_specs, out_shape):
        return pl.pallas_call(body, grid=(16,), in_specs=in_specs,
                              out_specs=out_specs, out_shape=out_shape)

    def _zn(z_ref, st_ref, g_ref, bb_ref):
        stv = st_ref[...]
        m = stv[0:1, :] / N
        v = stv[1:2, :] / N - m * m
        rstd = lax.rsqrt(v + 1e-5)
        return (z_ref[...] - m) * rstd * g_ref[...] + bb_ref[...]

    def _stats(i, z, st_ref):
        gid = i * RPT + lax.broadcasted_iota(jnp.int32, (RPT, 1), 0)
        zm = jnp.where(gid < N, z, 0.0)
        ssum = jnp.sum(zm, axis=0, keepdims=True)
        ssq = jnp.sum(zm * zm, axis=0, keepdims=True)
        upd = jnp.concatenate([ssum, ssq, jnp.zeros((6, H), F32)], axis=0)

        @pl.when(i == 0)
        def _():
            st_ref[...] = jnp.zeros((8, H), F32)

        st_ref[...] += upd

    def tc0(o0, xin):
        def body(o0_ref, x_ref, dis_o, u_o):
            deg = o0_ref[:, 0:1]
            dis = jnp.where(deg > 0, lax.rsqrt(jnp.maximum(deg, 1.0)), 0.0)
            dis_o[...] = dis
            ux = dis * x_ref[...]
            u_o[...] = jnp.concatenate(
                [ux, jnp.zeros((RPT, HW - 3), F32)], axis=1)

        return _call(body, [_rows(HW), _rows(3)],
                     [_rows(1), _rows(HW)],
                     [jax.ShapeDtypeStruct((NP, 1), F32),
                      jax.ShapeDtypeStruct((NP, HW), F32)])(o0, xin)

    def tc_m1(o0, dis):
        def body(o0_ref, dis_ref, u_o):
            w = -(dis_ref[...] * dis_ref[...])
            u_o[...] = w * o0_ref[...]

        return _call(body, [_rows(HW), _rows(1)], [_rows(HW)],
                     [jax.ShapeDtypeStruct((NP, HW), F32)])(o0, dis)[0]

    def tc_b1(xin, a0, q0, dis, W, b):
        def body(x_ref, a0_ref, q0_ref, dis_ref, w_ref, b_ref, z_o, st_o):
            i = pl.program_id(0)
            xb = x_ref[...]
            dis_ = dis_ref[...]
            t1 = -dis_ * a0_ref[:, :3]
            t2 = -2.0 * dis_ * q0_ref[:, :3] - xb
            Wm = w_ref[...]
            z = (jnp.dot(xb, Wm[0], preferred_element_type=F32)
                 + jnp.dot(t1, Wm[1], preferred_element_type=F32)
                 + jnp.dot(t2, Wm[2], preferred_element_type=F32)
                 + b_ref[...])
            z = jnp.where(z > 0, z, 0.01 * z)
            z_o[...] = z
            _stats(i, z, st_o)

        return _call(body,
                     [_rows(3), _rows(HW), _rows(HW), _rows(1),
                      _bcast((3, 3, H)), _bcast((1, H))],
                     [_rows(H), _bcast((8, H))],
                     [jax.ShapeDtypeStruct((NP, H), F32),
                      jax.ShapeDtypeStruct((8, H), F32)])(
                         xin, a0, q0, dis, W, b)

    def tc_a(z, st, g, bb, dis):
        def body(z_ref, st_ref, g_ref, bb_ref, dis_ref, u0_o, u1_o):
            zn = _zn(z_ref, st_ref, g_ref, bb_ref)
            un = dis_ref[...] * zn
            u0_o[...] = un[:, :HW]
            u1_o[...] = un[:, HW:]

        return _call(body,
                     [_rows(H), _bcast((8, H)), _bcast((1, H)), _bcast((1, H)),
                      _rows(1)],
                     [_rows(HW), _rows(HW)],
                     [jax.ShapeDtypeStruct((NP, HW), F32)] * 2)(
                         z, st, g, bb, dis)

    def tc_m(c0, c1, dis):
        def body(c0_ref, c1_ref, dis_ref, o0_ref, o1_ref):
            w = -(dis_ref[...] * dis_ref[...])
            o0_ref[...] = w * c0_ref[...]
            o1_ref[...] = w * c1_ref[...]

        return _call(body, [_rows(HW), _rows(HW), _rows(1)],
                     [_rows(HW), _rows(HW)],
                     [jax.ShapeDtypeStruct((NP, HW), F32)] * 2)(c0, c1, dis)

    def tc_b(z, st, g, bb, c0, c1, d0, d1, dis, W, b, leaky):
        def body(z_ref, st_ref, g_ref, bb_ref, c0_ref, c1_ref, d0_ref, d1_ref,
                 dis_ref, w_ref, b_ref, z_o, st_o):
            i = pl.program_id(0)
            zn = _zn(z_ref, st_ref, g_ref, bb_ref)
            dis_ = dis_ref[...]
            a1 = jnp.concatenate([c0_ref[...], c1_ref[...]], axis=1)
            a2 = jnp.concatenate([d0_ref[...], d1_ref[...]], axis=1)
            t1 = -dis_ * a1
            t2 = -2.0 * dis_ * a2 - zn
            Wm = w_ref[...]
            zz = (jnp.dot(zn, Wm[0], preferred_element_type=F32)
                  + jnp.dot(t1, Wm[1], preferred_element_type=F32)
                  + jnp.dot(t2, Wm[2], preferred_element_type=F32)
                  + b_ref[...])
            if leaky:
                zz = jnp.where(zz > 0, zz, 0.01 * zz)
            else:
                zz = jnp.maximum(zz, 0.0)
            z_o[...] = zz
            _stats(i, zz, st_o)

        return _call(body,
                     [_rows(H), _bcast((8, H)), _bcast((1, H)), _bcast((1, H)),
                      _rows(HW), _rows(HW), _rows(HW), _rows(HW),
                      _rows(1), _bcast((3, H, H)), _bcast((1, H))],
                     [_rows(H), _bcast((8, H))],
                     [jax.ShapeDtypeStruct((NP, H), F32),
                      jax.ShapeDtypeStruct((8, H), F32)])(
                         z, st, g, bb, c0, c1, d0, d1, dis, W, b)

    def tc_b4(z, st, g, bb, c0, c1, d0, d1, dis, W, b, lw, lb):
        def body(z_ref, st_ref, g_ref, bb_ref, c0_ref, c1_ref, d0_ref, d1_ref,
                 dis_ref, w_ref, b_ref, lw_ref, lb_ref, out_o):
            zn = _zn(z_ref, st_ref, g_ref, bb_ref)
            dis_ = dis_ref[...]
            a1 = jnp.concatenate([c0_ref[...], c1_ref[...]], axis=1)
            a2 = jnp.concatenate([d0_ref[...], d1_ref[...]], axis=1)
            t1 = -dis_ * a1
            t2 = -2.0 * dis_ * a2 - zn
            Wm = w_ref[...]
            h = (jnp.dot(zn, Wm[0], preferred_element_type=F32)
                 + jnp.dot(t1, Wm[1], preferred_element_type=F32)
                 + jnp.dot(t2, Wm[2], preferred_element_type=F32)
                 + b_ref[...])
            nrm = jnp.sqrt(jnp.sum(h * h, axis=1, keepdims=True))
            xr = h / jnp.maximum(nrm, 1e-12)
            out_o[...] = lax.dot_general(
                xr, lw_ref[...], (((1,), (1,)), ((), ())),
                preferred_element_type=F32) + lb_ref[...]

        return _call(body,
                     [_rows(H), _bcast((8, H)), _bcast((1, H)), _bcast((1, H)),
                      _rows(HW), _rows(HW), _rows(HW), _rows(HW),
                      _rows(1), _bcast((3, H, H)), _bcast((1, H)),
                      _bcast((3, H)), _bcast((1, 3))],
                     [_rows(3)],
                     [jax.ShapeDtypeStruct((N, 3), F32)])(
                         z, st, g, bb, c0, c1, d0, d1, dis, W, b, lw, lb)[0]


    x = x.astype(F32)
    src = edge_index[0]
    dst = edge_index[1]
    ones_u = jnp.zeros((NP, HW), F32).at[:, 0].set(1.0)
    zer32 = jnp.zeros((RPT, HW), F32)
    zeros_u = jnp.zeros((NP, HW), F32)
    b1 = conv1_b.reshape(1, H)
    b2 = conv2_b.reshape(1, H)
    b3 = conv3_b.reshape(1, H)
    b4 = conv4_b.reshape(1, H)
    g1, gb1 = bn1_g.reshape(1, H), bn1_b.reshape(1, H)
    g2, gb2 = bn2_g.reshape(1, H), bn2_b.reshape(1, H)
    g3, gb3 = bn3_g.reshape(1, H), bn3_b.reshape(1, H)
    lbr = lin_b.reshape(1, 3)

    sc_prop = _sc_prop()

    e_norm = jnp.stack([src, dst])

    def loop_body(k, carry):
        u0, u1, a0, a1v, z, st, dis, out = carry
        flg = jnp.where(k == 0, jnp.zeros((16,), jnp.int32),
                        jnp.zeros((16,), jnp.int32).at[0].set(1))
        o0, o1 = sc_prop(e_norm, flg, u0, u1, zer32)

        def s0(u0, u1, a0, a1v, z, st, dis, out):
            dis2, un0 = tc0(o0, x)
            return un0, u1, a0, a1v, z, st, dis2, out

        def s1(u0, u1, a0, a1v, z, st, dis, out):
            un0 = tc_m1(o0, dis)
            return un0, u1, o0, a1v, z, st, dis, out

        def s2(u0, u1, a0, a1v, z, st, dis, out):
            z1, st1 = tc_b1(x, a0, o0, dis, conv1_W, b1)
            n0, n1 = tc_a(z1, st1, g1, gb1, dis)
            return n0, n1, a0, a1v, z1, st1, dis, out

        def even(u0, u1, a0, a1v, z, st, dis, out):
            n0, n1 = tc_m(o0, o1, dis)
            return n0, n1, o0, o1, z, st, dis, out

        def mk_odd(g, gb, W, b, gn, gbn, leaky):
            def odd(u0, u1, a0, a1v, z, st, dis, out):
                z2, st2 = tc_b(z, st, g, gb, a0, a1v, o0, o1, dis, W, b, leaky)
                n0, n1 = tc_a(z2, st2, gn, gbn, dis)
                return n0, n1, a0, a1v, z2, st2, dis, out
            return odd

        def last(u0, u1, a0, a1v, z, st, dis, out):
            out2 = tc_b4(z, st, g3, gb3, a0, a1v, o0, o1, dis,
                         conv4_W, b4, lin_W, lbr)
            return u0, u1, a0, a1v, z, st, dis, out2

        branches = (s0, s1, s2,
                    even, mk_odd(g1, gb1, conv2_W, b2, g2, gb2, True),
                    even, mk_odd(g2, gb2, conv3_W, b3, g3, gb3, False),
                    even, last)
        return lax.switch(k, branches, u0, u1, a0, a1v, z, st, dis, out)

    zv = jnp.zeros((NP, HW), F32)
    carry = (ones_u, zeros_u, zv, zv,
             jnp.zeros((NP, H), F32), jnp.zeros((8, H), F32),
             jnp.zeros((NP, 1), F32), jnp.zeros((N, 3), F32))
    res = lax.fori_loop(0, 9, loop_body, carry)
    return res[7]
